# baseline (speedup 1.0000x reference)
import functools

import jax
import jax.numpy as jnp
from jax import lax
from jax.experimental import pallas as pl
from jax.experimental.pallas import tpu as pltpu
from jax.experimental.pallas import tpu_sc as plsc

N = 100000
E = 1600000
D = 16
NC = 2
NS = 16
NW = NC * NS
L = 16

E_TILE = E // NW
CH = 128
UNIT = 512
UCH = UNIT // CH
UNITS = 97
CODA = E_TILE - UNITS * UNIT
CODA_FULL = CODA // CH
TAIL = CODA - CODA_FULL * CH

N_PAD = 100096
ROWS_TILE = N_PAD // NS
ZCH = 184
ZREP = ROWS_TILE // ZCH

MINCH = 2048
E_SC = E // NS
MIN_FULL = E_SC // MINCH
MIN_TAIL = E_SC - MIN_FULL * MINCH

_I32_MAX = jnp.iinfo(jnp.int32).max


def _sc_body(edge_hbm, table_hbm, partial_hbm,
             acc, mins_sh,
             ebuf, gbuf, rowi, rowb_t,
             zbuf, minb, mvec, mmat,
             isem, gsem, ssem, msem, csem):
    c = lax.axis_index("c")
    s = lax.axis_index("s")
    wid = s * NC + c

    def zstore(k, carry):
        zbuf[k] = jnp.zeros((D,), jnp.float32)
        return carry
    lax.fori_loop(0, ZCH, zstore, 0)
    zd = []
    for j in range(ZREP):
        zd.append(pltpu.async_copy(
            zbuf, acc.at[pl.ds(s * ROWS_TILE + j * ZCH, ZCH)], csem))

    big = jnp.full((L,), _I32_MAX, jnp.int32)
    s_base = s * E_SC

    def scan_min(buf, nrows, m):
        full = nrows // (8 * L)
        def mbody(k, mm):
            base = k * (8 * L)
            for t in range(8):
                mm = jnp.minimum(mm, buf[pl.ds(base + t * L, L)])
            return mm
        m = lax.fori_loop(0, full, mbody, m)
        for r in range(full * 8, nrows // L):
            m = jnp.minimum(m, buf[pl.ds(r * L, L)])
        return m

    mdesc = [None, None]
    mdesc[0] = pltpu.async_copy(
        edge_hbm.at[pl.ds(s_base, MINCH)], minb.at[0], msem)
    m = big
    for ci in range(MIN_FULL):
        p = ci % 2
        mdesc[p].wait()
        if ci + 1 < MIN_FULL:
            mdesc[1 - p] = pltpu.async_copy(
                edge_hbm.at[pl.ds(s_base + (ci + 1) * MINCH, MINCH)],
                minb.at[1 - p], msem)
        elif MIN_TAIL:
            mdesc[1 - p] = pltpu.async_copy(
                edge_hbm.at[pl.ds(s_base + MIN_FULL * MINCH, MIN_TAIL)],
                minb.at[1 - p, pl.ds(0, MIN_TAIL)], msem)
        m = scan_min(minb.at[p], MINCH, m)
    if MIN_TAIL:
        p = MIN_FULL % 2
        mdesc[p].wait()
        m = scan_min(minb.at[p], MIN_TAIL, m)

    for j in range(8):
        mvec[j] = m
    pltpu.sync_copy(mvec, mins_sh.at[s])
    for d in zd:
        d.wait()
    plsc.subcore_barrier()

    pltpu.sync_copy(mins_sh, mmat)
    def rbody(i, mm):
        return jnp.minimum(mm, mmat[i, 0])
    m2 = lax.fori_loop(0, NS, rbody, big)
    lanes = jnp.arange(L, dtype=jnp.int32)
    for sh in (1, 2, 4, 8):
        m2 = jnp.minimum(m2, jnp.take(m2, (lanes + sh) % L))
    mn_vec = m2

    base0 = wid * E_TILE

    def fire_idx(u, p):
        base = base0 + u * UNIT
        pltpu.async_copy(edge_hbm.at[pl.ds(base, UNIT)],
                         ebuf.at[p, 0], isem[p])
        pltpu.async_copy(edge_hbm.at[pl.ds(E + base, UNIT)],
                         ebuf.at[p, 1], isem[p])

    def drain_idx(p):
        pltpu.make_async_copy(
            edge_hbm.at[pl.ds(0, UNIT)], ebuf.at[p, 0], isem[p]).wait()
        pltpu.make_async_copy(
            edge_hbm.at[pl.ds(0, UNIT)], ebuf.at[p, 1], isem[p]).wait()

    def fire_gathers(p):
        for k in range(UCH):
            pltpu.async_copy(
                table_hbm.at[ebuf.at[p, 1, pl.ds(k * CH, CH)]],
                gbuf.at[p, k], gsem[p])

    def drain_gathers(p):
        for k in range(UCH):
            pltpu.make_async_copy(
                table_hbm.at[pl.ds(0, CH)], gbuf.at[p, k], gsem[p]).wait()

    def vsub(p):
        for k in range(UCH):
            for t in range(CH // L):
                rowi[p, k, pl.ds(t * L, L)] = (
                    ebuf[p, 0, pl.ds(k * CH + t * L, L)] - mn_vec)

    def fire_scatters(p):
        for k in range(UCH):
            pltpu.async_copy(
                gbuf.at[p, k], acc.at[rowi.at[p, k]], ssem[p], add=True)

    def drain_scatters(p):
        for k in range(UCH):
            pltpu.make_async_copy(
                table_hbm.at[pl.ds(0, CH)], gbuf.at[p, k], ssem[p]).wait()

    def half(ufire, p):
        q = 1 - p
        @pl.when(ufire >= 2)
        def _():
            drain_scatters(p)
        drain_idx(p)
        fire_gathers(p)
        drain_gathers(q)
        vsub(q)
        @pl.when(ufire + 1 <= UNITS - 1)
        def _():
            fire_idx(ufire + 1, q)
        fire_scatters(q)

    fire_idx(jnp.int32(0), 0)
    fire_idx(jnp.int32(1), 1)
    drain_idx(0)
    fire_gathers(0)

    def pipe_body(j, carry):
        half(2 * j + 1, 1)
        half(2 * j + 2, 0)
        return carry
    lax.fori_loop(0, (UNITS - 1) // 2, pipe_body, 0)

    drain_gathers(0)
    vsub(0)
    fire_scatters(0)
    drain_scatters(1)
    basec = base0 + UNITS * UNIT
    pltpu.sync_copy(edge_hbm.at[pl.ds(basec, CODA)],
                    ebuf.at[1, 0, pl.ds(0, CODA)])
    pltpu.sync_copy(edge_hbm.at[pl.ds(E + basec, CODA)],
                    ebuf.at[1, 1, pl.ds(0, CODA)])
    cd = []
    for k in range(CODA_FULL):
        cd.append(pltpu.async_copy(
            table_hbm.at[ebuf.at[1, 1, pl.ds(k * CH, CH)]],
            gbuf.at[1, k], csem))
    cd.append(pltpu.async_copy(
        table_hbm.at[ebuf.at[1, 1, pl.ds(CODA_FULL * CH, TAIL)]],
        gbuf.at[1, CODA_FULL, pl.ds(0, TAIL)], csem))
    for d in cd:
        d.wait()
    for k in range(CODA_FULL):
        for t in range(CH // L):
            rowi[1, k, pl.ds(t * L, L)] = (
                ebuf[1, 0, pl.ds(k * CH + t * L, L)] - mn_vec)
    for t in range(TAIL // L):
        rowb_t[pl.ds(t * L, L)] = (
            ebuf[1, 0, pl.ds(CODA_FULL * CH + t * L, L)] - mn_vec)
    for k in range(CODA_FULL):
        pltpu.sync_copy(gbuf.at[1, k], acc.at[rowi.at[1, k]], add=True)
    pltpu.sync_copy(gbuf.at[1, CODA_FULL, pl.ds(0, TAIL)],
                    acc.at[rowb_t], add=True)
    drain_scatters(0)

    plsc.subcore_barrier()

    pltpu.sync_copy(acc.at[pl.ds(s * ROWS_TILE, ROWS_TILE)],
                    partial_hbm.at[c, pl.ds(s * ROWS_TILE, ROWS_TILE)])


_sc_lookup = functools.partial(
    pl.kernel,
    out_type=jax.ShapeDtypeStruct((NC, N_PAD, D), jnp.float32),
    mesh=plsc.VectorSubcoreMesh(core_axis_name="c", subcore_axis_name="s"),
    scratch_types=[
        pltpu.VMEM_SHARED((N_PAD, D), jnp.float32),
        pltpu.VMEM_SHARED((NS, 8, L), jnp.int32),
        pltpu.VMEM((2, 2, UNIT), jnp.int32),
        pltpu.VMEM((2, UCH, CH, D), jnp.float32),
        pltpu.VMEM((2, UCH, CH), jnp.int32),
        pltpu.VMEM((TAIL,), jnp.int32),
        pltpu.VMEM((ZCH, D), jnp.float32),
        pltpu.VMEM((2, MINCH), jnp.int32),
        pltpu.VMEM((8, L), jnp.int32),
        pltpu.VMEM((NS, 8, L), jnp.int32),
        [pltpu.SemaphoreType.DMA] * 2,
        [pltpu.SemaphoreType.DMA] * 2,
        [pltpu.SemaphoreType.DMA] * 2,
        pltpu.SemaphoreType.DMA,
        pltpu.SemaphoreType.DMA,
    ],
    compiler_params=pltpu.CompilerParams(use_tc_tiling_on_sc=False),
)(_sc_body)


M_TILE = N // NW
MCH = 512
M_FULL = M_TILE // MCH
M_TAIL = M_TILE - M_FULL * MCH


def _merge_body(partial_hbm, bias_hbm, out_hbm, pbuf, obuf, bbuf, lsem, osem):
    c = lax.axis_index("c")
    s = lax.axis_index("s")
    wid = s * NC + c
    base = wid * M_TILE

    pltpu.sync_copy(bias_hbm, bbuf)
    bvec = bbuf[...]

    sizes = [MCH] * M_FULL + ([M_TAIL] if M_TAIL else [])

    def fire_load(ci, p):
        off = base + ci * MCH
        n = sizes[ci]
        for src in range(2):
            pltpu.async_copy(
                partial_hbm.at[src, pl.ds(off, n)],
                pbuf.at[p, src, pl.ds(0, n)], lsem[p])

    ld = {}
    st = {}
    ld[0] = fire_load(0, 0)
    for ci in range(len(sizes)):
        p = ci % 2
        n = sizes[ci]
        for src in range(2):
            pltpu.make_async_copy(
                partial_hbm.at[0, pl.ds(0, n)],
                pbuf.at[p, src, pl.ds(0, n)], lsem[p]).wait()
        if ci + 1 < len(sizes):
            fire_load(ci + 1, 1 - p)
        if ci >= 2:
            pltpu.make_async_copy(
                out_hbm.at[pl.ds(0, sizes[ci - 2])],
                obuf.at[p, pl.ds(0, sizes[ci - 2])], osem[p]).wait()
        def abody(i, carry):
            obuf[p, i] = pbuf[p, 0, i] + pbuf[p, 1, i] + bvec
            return carry
        lax.fori_loop(0, n, abody, 0)
        pltpu.async_copy(
            obuf.at[p, pl.ds(0, n)],
            out_hbm.at[pl.ds(base + ci * MCH, n)], osem[p])
    for ci in (len(sizes) - 2, len(sizes) - 1):
        if ci >= 0:
            p = ci % 2
            pltpu.make_async_copy(
                out_hbm.at[pl.ds(0, sizes[ci])],
                obuf.at[p, pl.ds(0, sizes[ci])], osem[p]).wait()


_sc_merge = functools.partial(
    pl.kernel,
    out_type=jax.ShapeDtypeStruct((N, D), jnp.float32),
    mesh=plsc.VectorSubcoreMesh(core_axis_name="c", subcore_axis_name="s"),
    scratch_types=[
        pltpu.VMEM((2, 2, MCH, D), jnp.float32),
        pltpu.VMEM((2, MCH, D), jnp.float32),
        pltpu.VMEM((D,), jnp.float32),
        [pltpu.SemaphoreType.DMA] * 2,
        [pltpu.SemaphoreType.DMA] * 2,
    ],
    compiler_params=pltpu.CompilerParams(use_tc_tiling_on_sc=False),
)(_merge_body)


def kernel(edge_index, W_weight, W_bias):
    table = W_weight.T
    edge_flat = edge_index.reshape(2 * E)
    partial = _sc_lookup(edge_flat, table)
    return _sc_merge(partial, W_bias)


# speedup vs baseline: 1.0380x; 1.0380x over previous
import functools

import jax
import jax.numpy as jnp
from jax import lax
from jax.experimental import pallas as pl
from jax.experimental.pallas import tpu as pltpu
from jax.experimental.pallas import tpu_sc as plsc

N = 100000
E = 1600000
D = 16
NC = 2
NS = 16
NW = NC * NS
L = 16

E_TILE = E // NW
CH = 128
UNIT = 512
UCH = UNIT // CH
UNITS = 97
CODA = E_TILE - UNITS * UNIT
CODA_FULL = CODA // CH
TAIL = CODA - CODA_FULL * CH

N_PAD = 100096
ROWS_TILE = N_PAD // NS
ZCH = 184
ZREP = ROWS_TILE // ZCH

MINCH = 2048
E_SC = E // NS
MIN_FULL = E_SC // MINCH
MIN_TAIL = E_SC - MIN_FULL * MINCH

_I32_MAX = jnp.iinfo(jnp.int32).max


def _sc_body(edge_hbm, table_hbm, partial_hbm,
             acc, mins_sh,
             ebuf, gbuf, rowi, rowb_t,
             zbuf, minb, mvec, mmat,
             isem, gsem, ssem, msem, csem):
    c = lax.axis_index("c")
    s = lax.axis_index("s")
    wid = s * NC + c

    def zstore(k, carry):
        zbuf[k] = jnp.zeros((D,), jnp.float32)
        return carry
    lax.fori_loop(0, ZCH, zstore, 0)
    zd = []
    for j in range(ZREP):
        zd.append(pltpu.async_copy(
            zbuf, acc.at[pl.ds(s * ROWS_TILE + j * ZCH, ZCH)], csem))

    big = jnp.full((L,), _I32_MAX, jnp.int32)
    s_base = s * E_SC

    def scan_min(buf, nrows, m):
        full = nrows // (8 * L)
        def mbody(k, mm):
            base = k * (8 * L)
            for t in range(8):
                mm = jnp.minimum(mm, buf[pl.ds(base + t * L, L)])
            return mm
        m = lax.fori_loop(0, full, mbody, m)
        for r in range(full * 8, nrows // L):
            m = jnp.minimum(m, buf[pl.ds(r * L, L)])
        return m

    mdesc = [None, None]
    mdesc[0] = pltpu.async_copy(
        edge_hbm.at[0, pl.ds(s_base, MINCH)], minb.at[0], msem)
    m = big
    for ci in range(MIN_FULL):
        p = ci % 2
        mdesc[p].wait()
        if ci + 1 < MIN_FULL:
            mdesc[1 - p] = pltpu.async_copy(
                edge_hbm.at[0, pl.ds(s_base + (ci + 1) * MINCH, MINCH)],
                minb.at[1 - p], msem)
        elif MIN_TAIL:
            mdesc[1 - p] = pltpu.async_copy(
                edge_hbm.at[0, pl.ds(s_base + MIN_FULL * MINCH, MIN_TAIL)],
                minb.at[1 - p, pl.ds(0, MIN_TAIL)], msem)
        m = scan_min(minb.at[p], MINCH, m)
    if MIN_TAIL:
        p = MIN_FULL % 2
        mdesc[p].wait()
        m = scan_min(minb.at[p], MIN_TAIL, m)

    for j in range(8):
        mvec[j] = m
    pltpu.sync_copy(mvec, mins_sh.at[s])
    for d in zd:
        d.wait()
    plsc.subcore_barrier()

    pltpu.sync_copy(mins_sh, mmat)
    def rbody(i, mm):
        return jnp.minimum(mm, mmat[i, 0])
    m2 = lax.fori_loop(0, NS, rbody, big)
    lanes = jnp.arange(L, dtype=jnp.int32)
    for sh in (1, 2, 4, 8):
        m2 = jnp.minimum(m2, jnp.take(m2, (lanes + sh) % L))
    mn_vec = m2

    base0 = wid * E_TILE

    def fire_idx(u, p):
        pltpu.async_copy(edge_hbm.at[:, pl.ds(base0 + u * UNIT, UNIT)],
                         ebuf.at[p], isem[p])

    def drain_idx(p):
        pltpu.make_async_copy(
            edge_hbm.at[:, pl.ds(0, UNIT)], ebuf.at[p], isem[p]).wait()

    def fire_gathers(p):
        for k in range(UCH):
            pltpu.async_copy(
                table_hbm.at[ebuf.at[p, 1, pl.ds(k * CH, CH)]],
                gbuf.at[p, k], gsem[p])

    def drain_gathers(p):
        for k in range(UCH):
            pltpu.make_async_copy(
                table_hbm.at[pl.ds(0, CH)], gbuf.at[p, k], gsem[p]).wait()

    def vsub(p):
        for k in range(UCH):
            for t in range(CH // L):
                rowi[p, k, pl.ds(t * L, L)] = (
                    ebuf[p, 0, pl.ds(k * CH + t * L, L)] - mn_vec)

    def fire_scatters(p):
        for k in range(UCH):
            pltpu.async_copy(
                gbuf.at[p, k], acc.at[rowi.at[p, k]], ssem[p], add=True)

    def drain_scatters(p):
        for k in range(UCH):
            pltpu.make_async_copy(
                table_hbm.at[pl.ds(0, CH)], gbuf.at[p, k], ssem[p]).wait()

    def half(ufire, p):
        q = 1 - p
        @pl.when(ufire >= 2)
        def _():
            drain_scatters(p)
        drain_idx(p)
        fire_gathers(p)
        drain_gathers(q)
        vsub(q)
        @pl.when(ufire + 1 <= UNITS - 1)
        def _():
            fire_idx(ufire + 1, q)
        fire_scatters(q)

    fire_idx(jnp.int32(0), 0)
    fire_idx(jnp.int32(1), 1)
    drain_idx(0)
    fire_gathers(0)

    def pipe_body(j, carry):
        half(2 * j + 1, 1)
        half(2 * j + 2, 0)
        return carry
    lax.fori_loop(0, (UNITS - 1) // 2, pipe_body, 0)

    drain_gathers(0)
    vsub(0)
    fire_scatters(0)
    drain_scatters(1)
    basec = base0 + UNITS * UNIT
    pltpu.sync_copy(edge_hbm.at[:, pl.ds(basec, CODA)],
                    ebuf.at[1, :, pl.ds(0, CODA)])
    cd = []
    for k in range(CODA_FULL):
        cd.append(pltpu.async_copy(
            table_hbm.at[ebuf.at[1, 1, pl.ds(k * CH, CH)]],
            gbuf.at[1, k], csem))
    cd.append(pltpu.async_copy(
        table_hbm.at[ebuf.at[1, 1, pl.ds(CODA_FULL * CH, TAIL)]],
        gbuf.at[1, CODA_FULL, pl.ds(0, TAIL)], csem))
    for d in cd:
        d.wait()
    for k in range(CODA_FULL):
        for t in range(CH // L):
            rowi[1, k, pl.ds(t * L, L)] = (
                ebuf[1, 0, pl.ds(k * CH + t * L, L)] - mn_vec)
    for t in range(TAIL // L):
        rowb_t[pl.ds(t * L, L)] = (
            ebuf[1, 0, pl.ds(CODA_FULL * CH + t * L, L)] - mn_vec)
    for k in range(CODA_FULL):
        pltpu.sync_copy(gbuf.at[1, k], acc.at[rowi.at[1, k]], add=True)
    pltpu.sync_copy(gbuf.at[1, CODA_FULL, pl.ds(0, TAIL)],
                    acc.at[rowb_t], add=True)
    drain_scatters(0)

    plsc.subcore_barrier()

    pltpu.sync_copy(acc.at[pl.ds(s * ROWS_TILE, ROWS_TILE)],
                    partial_hbm.at[c, pl.ds(s * ROWS_TILE, ROWS_TILE)])


_sc_lookup = functools.partial(
    pl.kernel,
    out_type=jax.ShapeDtypeStruct((NC, N_PAD, D), jnp.float32),
    mesh=plsc.VectorSubcoreMesh(core_axis_name="c", subcore_axis_name="s"),
    scratch_types=[
        pltpu.VMEM_SHARED((N_PAD, D), jnp.float32),
        pltpu.VMEM_SHARED((NS, 8, L), jnp.int32),
        pltpu.VMEM((2, 2, UNIT), jnp.int32),
        pltpu.VMEM((2, UCH, CH, D), jnp.float32),
        pltpu.VMEM((2, UCH, CH), jnp.int32),
        pltpu.VMEM((TAIL,), jnp.int32),
        pltpu.VMEM((ZCH, D), jnp.float32),
        pltpu.VMEM((2, MINCH), jnp.int32),
        pltpu.VMEM((8, L), jnp.int32),
        pltpu.VMEM((NS, 8, L), jnp.int32),
        [pltpu.SemaphoreType.DMA] * 2,
        [pltpu.SemaphoreType.DMA] * 2,
        [pltpu.SemaphoreType.DMA] * 2,
        pltpu.SemaphoreType.DMA,
        pltpu.SemaphoreType.DMA,
    ],
    compiler_params=pltpu.CompilerParams(use_tc_tiling_on_sc=False),
)(_sc_body)


_NROWS = (N * D) // 128


def _merge_body(p_ref, b_ref, o_ref):
    o_ref[...] = p_ref[0, :_NROWS] + p_ref[1, :_NROWS] + b_ref[...]


_merge = pl.pallas_call(
    _merge_body,
    out_shape=jax.ShapeDtypeStruct((_NROWS, 128), jnp.float32),
)


def kernel(edge_index, W_weight, W_bias):
    table = W_weight.T
    partial = _sc_lookup(edge_index, table)
    p = partial.reshape(NC, (N_PAD * D) // 128, 128)
    b = jnp.tile(W_bias, 128 // D)
    out = _merge(p, b)
    return out.reshape(N, D)


# speedup vs baseline: 1.0748x; 1.0355x over previous
import functools

import jax
import jax.numpy as jnp
from jax import lax
from jax.experimental import pallas as pl
from jax.experimental.pallas import tpu as pltpu
from jax.experimental.pallas import tpu_sc as plsc

N = 100000
E = 1600000
D = 16
NC = 2
NS = 16
NW = NC * NS
L = 16

E_TILE = E // NW
CH = 128
UNIT = 512
UCH = UNIT // CH
UNITS = 97
CODA = E_TILE - UNITS * UNIT
CODA_FULL = CODA // CH
TAIL = CODA - CODA_FULL * CH

N_PAD = 100096
ROWS_TILE = N_PAD // NS
ZCH = 184
ZREP = ROWS_TILE // ZCH

MINCH = 3072
E_SC = E // NS
MIN_FULL = E_SC // MINCH
MIN_TAIL = E_SC - MIN_FULL * MINCH

_I32_MAX = jnp.iinfo(jnp.int32).max


def _sc_body(edge_hbm, table_hbm, partial_hbm,
             acc, mins_sh,
             ebuf, gbuf, rowi, rowb_t,
             zbuf, minb, mvec, mmat,
             isem, gsem, ssem, msem, csem):
    c = lax.axis_index("c")
    s = lax.axis_index("s")
    wid = s * NC + c

    def zstore(k, carry):
        zbuf[k] = jnp.zeros((D,), jnp.float32)
        return carry
    lax.fori_loop(0, ZCH, zstore, 0)
    zd = []
    for j in range(ZREP):
        zd.append(pltpu.async_copy(
            zbuf, acc.at[pl.ds(s * ROWS_TILE + j * ZCH, ZCH)], csem))

    big = jnp.full((L,), _I32_MAX, jnp.int32)
    s_base = s * E_SC

    def scan_min(buf, nrows, m):
        full = nrows // (8 * L)
        def mbody(k, mm):
            base = k * (8 * L)
            for t in range(8):
                mm = jnp.minimum(mm, buf[pl.ds(base + t * L, L)])
            return mm
        m = lax.fori_loop(0, full, mbody, m)
        for r in range(full * 8, nrows // L):
            m = jnp.minimum(m, buf[pl.ds(r * L, L)])
        return m

    mdesc = [None, None]
    mdesc[0] = pltpu.async_copy(
        edge_hbm.at[0, pl.ds(s_base, MINCH)], minb.at[0], msem)
    m = big
    for ci in range(MIN_FULL):
        p = ci % 2
        mdesc[p].wait()
        if ci + 1 < MIN_FULL:
            mdesc[1 - p] = pltpu.async_copy(
                edge_hbm.at[0, pl.ds(s_base + (ci + 1) * MINCH, MINCH)],
                minb.at[1 - p], msem)
        elif MIN_TAIL:
            mdesc[1 - p] = pltpu.async_copy(
                edge_hbm.at[0, pl.ds(s_base + MIN_FULL * MINCH, MIN_TAIL)],
                minb.at[1 - p, pl.ds(0, MIN_TAIL)], msem)
        m = scan_min(minb.at[p], MINCH, m)
    if MIN_TAIL:
        p = MIN_FULL % 2
        mdesc[p].wait()
        m = scan_min(minb.at[p], MIN_TAIL, m)

    for j in range(8):
        mvec[j] = m
    pltpu.sync_copy(mvec, mins_sh.at[s])
    for d in zd:
        d.wait()
    plsc.subcore_barrier()

    pltpu.sync_copy(mins_sh, mmat)
    def rbody(i, mm):
        return jnp.minimum(mm, mmat[i, 0])
    m2 = lax.fori_loop(0, NS, rbody, big)
    lanes = jnp.arange(L, dtype=jnp.int32)
    for sh in (1, 2, 4, 8):
        m2 = jnp.minimum(m2, jnp.take(m2, (lanes + sh) % L))
    mn_vec = m2

    base0 = wid * E_TILE

    def fire_idx(u, p):
        pltpu.async_copy(edge_hbm.at[:, pl.ds(base0 + u * UNIT, UNIT)],
                         ebuf.at[p], isem[p])

    def drain_idx(p):
        pltpu.make_async_copy(
            edge_hbm.at[:, pl.ds(0, UNIT)], ebuf.at[p], isem[p]).wait()

    def fire_gathers(p):
        for k in range(UCH):
            pltpu.async_copy(
                table_hbm.at[ebuf.at[p, 1, pl.ds(k * CH, CH)]],
                gbuf.at[p, k], gsem[p])

    def drain_gathers(p):
        for k in range(UCH):
            pltpu.make_async_copy(
                table_hbm.at[pl.ds(0, CH)], gbuf.at[p, k], gsem[p]).wait()

    def vsub(p):
        for k in range(UCH):
            for t in range(CH // L):
                rowi[p, k, pl.ds(t * L, L)] = (
                    ebuf[p, 0, pl.ds(k * CH + t * L, L)] - mn_vec)

    def fire_scatters(p):
        for k in range(UCH):
            pltpu.async_copy(
                gbuf.at[p, k], acc.at[rowi.at[p, k]], ssem[p], add=True)

    def drain_scatters(p):
        for k in range(UCH):
            pltpu.make_async_copy(
                table_hbm.at[pl.ds(0, CH)], gbuf.at[p, k], ssem[p]).wait()

    def half(ufire, p):
        q = 1 - p
        @pl.when(ufire >= 2)
        def _():
            drain_scatters(p)
        drain_idx(p)
        fire_gathers(p)
        drain_gathers(q)
        vsub(q)
        @pl.when(ufire + 1 <= UNITS - 1)
        def _():
            fire_idx(ufire + 1, q)
        fire_scatters(q)

    fire_idx(jnp.int32(0), 0)
    fire_idx(jnp.int32(1), 1)
    drain_idx(0)
    fire_gathers(0)

    def pipe_body(j, carry):
        half(2 * j + 1, 1)
        half(2 * j + 2, 0)
        return carry
    lax.fori_loop(0, (UNITS - 1) // 2, pipe_body, 0)

    drain_gathers(0)
    vsub(0)
    fire_scatters(0)
    drain_scatters(1)
    basec = base0 + UNITS * UNIT
    pltpu.sync_copy(edge_hbm.at[:, pl.ds(basec, CODA)],
                    ebuf.at[1, :, pl.ds(0, CODA)])
    cd = []
    for k in range(CODA_FULL):
        cd.append(pltpu.async_copy(
            table_hbm.at[ebuf.at[1, 1, pl.ds(k * CH, CH)]],
            gbuf.at[1, k], csem))
    cd.append(pltpu.async_copy(
        table_hbm.at[ebuf.at[1, 1, pl.ds(CODA_FULL * CH, TAIL)]],
        gbuf.at[1, CODA_FULL, pl.ds(0, TAIL)], csem))
    for d in cd:
        d.wait()
    for k in range(CODA_FULL):
        for t in range(CH // L):
            rowi[1, k, pl.ds(t * L, L)] = (
                ebuf[1, 0, pl.ds(k * CH + t * L, L)] - mn_vec)
    for t in range(TAIL // L):
        rowb_t[pl.ds(t * L, L)] = (
            ebuf[1, 0, pl.ds(CODA_FULL * CH + t * L, L)] - mn_vec)
    for k in range(CODA_FULL):
        pltpu.sync_copy(gbuf.at[1, k], acc.at[rowi.at[1, k]], add=True)
    pltpu.sync_copy(gbuf.at[1, CODA_FULL, pl.ds(0, TAIL)],
                    acc.at[rowb_t], add=True)
    drain_scatters(0)

    plsc.subcore_barrier()

    pltpu.sync_copy(acc.at[pl.ds(s * ROWS_TILE, ROWS_TILE)],
                    partial_hbm.at[c, pl.ds(s * ROWS_TILE, ROWS_TILE)])


_sc_lookup = functools.partial(
    pl.kernel,
    out_type=jax.ShapeDtypeStruct((NC, N_PAD, D), jnp.float32),
    mesh=plsc.VectorSubcoreMesh(core_axis_name="c", subcore_axis_name="s"),
    scratch_types=[
        pltpu.VMEM_SHARED((N_PAD, D), jnp.float32),
        pltpu.VMEM_SHARED((NS, 8, L), jnp.int32),
        pltpu.VMEM((2, 2, UNIT), jnp.int32),
        pltpu.VMEM((2, UCH, CH, D), jnp.float32),
        pltpu.VMEM((2, UCH, CH), jnp.int32),
        pltpu.VMEM((TAIL,), jnp.int32),
        pltpu.VMEM((ZCH, D), jnp.float32),
        pltpu.VMEM((2, MINCH), jnp.int32),
        pltpu.VMEM((8, L), jnp.int32),
        pltpu.VMEM((NS, 8, L), jnp.int32),
        [pltpu.SemaphoreType.DMA] * 2,
        [pltpu.SemaphoreType.DMA] * 2,
        [pltpu.SemaphoreType.DMA] * 2,
        pltpu.SemaphoreType.DMA,
        pltpu.SemaphoreType.DMA,
    ],
    compiler_params=pltpu.CompilerParams(use_tc_tiling_on_sc=False),
)(_sc_body)


_NROWS = (N * D) // 128


def _merge_body(p_ref, b_ref, o_ref):
    o_ref[...] = p_ref[0, :_NROWS] + p_ref[1, :_NROWS] + b_ref[...]


_merge = pl.pallas_call(
    _merge_body,
    out_shape=jax.ShapeDtypeStruct((_NROWS, 128), jnp.float32),
)


def kernel(edge_index, W_weight, W_bias):
    table = W_weight.T
    partial = _sc_lookup(edge_index, table)
    p = partial.reshape(NC, (N_PAD * D) // 128, 128)
    b = jnp.tile(W_bias, 128 // D)
    out = _merge(p, b)
    return out.reshape(N, D)
